# baseline (speedup 1.0000x reference)
import functools

import jax
import jax.numpy as jnp
from jax import lax
from jax.experimental import pallas as pl
from jax.experimental.pallas import tpu as pltpu
from jax.experimental.pallas import tpu_sc as plsc

NC = 2
NS = 16
NW = NC * NS

SCHUNK = 50
SUBCH = 200
NPAD = 10240


def _sc_degree(ei5):
    npt = NPAD // NS

    mesh = plsc.VectorSubcoreMesh(core_axis_name="c", subcore_axis_name="s")

    @functools.partial(
        pl.kernel,
        out_type=jax.ShapeDtypeStruct((NC, NPAD), jnp.float32),
        mesh=mesh,
        scratch_types=[
            pltpu.VMEM((SUBCH, 2, SCHUNK), jnp.int32),
            pltpu.VMEM((128,), jnp.float32),
            pltpu.VMEM((npt,), jnp.float32),
            pltpu.VMEM_SHARED((NPAD,), jnp.float32),
            pltpu.SemaphoreType.DMA,
        ],
    )
    def k(ei_hbm, out_hbm, idx_v, ones_v, stage_v, hist_sh, sem):
        c = lax.axis_index("c")
        s = lax.axis_index("s")
        wid = c * NS + s

        def fill_ones(i, _):
            ones_v[pl.ds(i * 16, 16)] = jnp.full((16,), 1.0, jnp.float32)
            return 0

        lax.fori_loop(0, 8, fill_ones, 0)

        def zero_stage(i, _):
            stage_v[pl.ds(i * 16, 16)] = jnp.zeros((16,), jnp.float32)
            return 0

        lax.fori_loop(0, npt // 16, zero_stage, 0)

        pltpu.sync_copy(ei_hbm.at[wid], idx_v)
        pltpu.sync_copy(stage_v, hist_sh.at[pl.ds(s * npt, npt)])
        plsc.subcore_barrier()

        ones = ones_v.at[pl.ds(0, SCHUNK)]

        def fire(q, _):
            pltpu.async_copy(ones, hist_sh.at[idx_v.at[q, 1]], sem, add=True)
            return 0

        lax.fori_loop(0, SUBCH, fire, 0)

        def drain(q, _):
            pltpu.make_async_copy(ones, hist_sh.at[idx_v.at[q, 1]],
                                  sem).wait()
            return 0

        lax.fori_loop(0, SUBCH, drain, 0)
        plsc.subcore_barrier()

        pltpu.sync_copy(hist_sh.at[pl.ds(s * npt, npt)], stage_v)
        pltpu.sync_copy(stage_v, out_hbm.at[c, pl.ds(s * npt, npt)])

    return k(ei5)


def _sc_apply(z, ei5):
    N, D = z.shape
    rpt = NPAD // NS
    RC = 64
    n_rc = rpt // RC
    NB = 5
    NSLOT = 10
    n_outer = SUBCH // NSLOT

    mesh = plsc.VectorSubcoreMesh(core_axis_name="c", subcore_axis_name="s")

    @functools.partial(
        pl.kernel,
        out_type=jax.ShapeDtypeStruct((NC, NPAD, D), jnp.float32),
        mesh=mesh,
        scratch_types=[
            pltpu.VMEM((NSLOT, 2, SCHUNK), jnp.int32),
            [pltpu.VMEM((SCHUNK, D), jnp.float32) for _ in range(NB)],
            pltpu.VMEM((RC, D), jnp.float32),
            pltpu.VMEM_SHARED((NPAD, D), jnp.float32),
            [pltpu.SemaphoreType.DMA for _ in range(NSLOT)],
            [pltpu.SemaphoreType.DMA for _ in range(NB)],
            [pltpu.SemaphoreType.DMA for _ in range(NB)],
            pltpu.SemaphoreType.DMA,
        ],
    )
    def k(z_hbm, ei_hbm, out_hbm, idx_v, rows, stage_v, acc_sh,
          isem, gsem, ssem, osem):
        c = lax.axis_index("c")
        s = lax.axis_index("s")
        wid = c * NS + s

        def load_idx(m, j):
            pltpu.async_copy(ei_hbm.at[wid, m], idx_v.at[j], isem[j])

        def wait_idx(m, j):
            pltpu.make_async_copy(
                ei_hbm.at[wid, m], idx_v.at[j], isem[j]).wait()

        def gather(m, b, j):
            pltpu.async_copy(z_hbm.at[idx_v.at[j, 0]], rows[b], gsem[b])

        def wait_gather(b, j):
            pltpu.make_async_copy(
                z_hbm.at[idx_v.at[j, 0]], rows[b], gsem[b]).wait()

        def scatter(b, j):
            pltpu.async_copy(
                rows[b], acc_sh.at[idx_v.at[j, 1]], ssem[b], add=True)

        def wait_scatter(b, j):
            pltpu.make_async_copy(
                rows[b], acc_sh.at[idx_v.at[j, 1]], ssem[b]).wait()

        for j in range(7):
            load_idx(j, j)

        def zrow(i, _):
            def zcol(j, _):
                stage_v[i, pl.ds(j * 16, 16)] = jnp.zeros((16,), jnp.float32)
                return 0

            lax.fori_loop(0, D // 16, zcol, 0)
            return 0

        lax.fori_loop(0, RC, zrow, 0)

        def zacc(i, _):
            pltpu.async_copy(
                stage_v, acc_sh.at[pl.ds(s * rpt + i * RC, RC)], osem)
            return 0

        lax.fori_loop(0, n_rc, zacc, 0)

        wait_idx(0, 0)
        gather(0, 0, 0)
        wait_idx(1, 1)
        gather(1, 1, 1)

        def zacc_drain(i, _):
            pltpu.make_async_copy(
                stage_v, acc_sh.at[pl.ds(s * rpt + i * RC, RC)], osem).wait()
            return 0

        lax.fori_loop(0, n_rc, zacc_drain, 0)
        plsc.subcore_barrier()

        def outer(i, _):
            for t in range(NSLOT):
                b = t % NB
                wait_gather(b, t)
                scatter(b, t)

                bp = (t - 3) % NB
                jp = (t - 3) % NSLOT
                if t >= 3:
                    wait_scatter(bp, jp)
                else:

                    @pl.when(i > 0)
                    def _():
                        wait_scatter(bp, jp)

                jn = (t + 7) % NSLOT
                if t < 3:
                    load_idx(i * NSLOT + t + 7, jn)
                else:

                    @pl.when(i < n_outer - 1)
                    def _():
                        load_idx(i * NSLOT + t + 7, jn)

                bg = (t + 2) % NB
                jg = (t + 2) % NSLOT
                if t < NSLOT - 2:
                    wait_idx(i * NSLOT + t + 2, jg)
                    gather(i * NSLOT + t + 2, bg, jg)
                else:

                    @pl.when(i < n_outer - 1)
                    def _():
                        wait_idx(i * NSLOT + t + 2, jg)
                        gather(i * NSLOT + t + 2, bg, jg)

            return 0

        lax.fori_loop(0, n_outer, outer, 0)
        wait_scatter((SUBCH - 3) % NB, (SUBCH - 3) % NSLOT)
        wait_scatter((SUBCH - 2) % NB, (SUBCH - 2) % NSLOT)
        wait_scatter((SUBCH - 1) % NB, (SUBCH - 1) % NSLOT)
        plsc.subcore_barrier()

        def ocopy(i, _):
            pltpu.async_copy(
                acc_sh.at[pl.ds(s * rpt + i * RC, RC)],
                out_hbm.at[c, pl.ds(s * rpt + i * RC, RC)], osem)
            return 0

        lax.fori_loop(0, n_rc, ocopy, 0)

        def ocopy_drain(i, _):
            pltpu.make_async_copy(
                acc_sh.at[pl.ds(s * rpt + i * RC, RC)],
                out_hbm.at[c, pl.ds(s * rpt + i * RC, RC)], osem).wait()
            return 0

        lax.fori_loop(0, n_rc, ocopy_drain, 0)

    return k(z, ei5)


_BN = 1024


def _tc_prescale(x, degp):
    N, D = x.shape
    grid = (NPAD // _BN,)

    def body(x_ref, degp_ref, z_ref, dinv_ref):
        deg = degp_ref[0, :] + degp_ref[1, :] + 1.0
        dinv = lax.rsqrt(deg)
        dinv_ref[...] = dinv
        z_ref[...] = x_ref[...] * dinv[:, None]

    return pl.pallas_call(
        body,
        grid=grid,
        in_specs=[
            pl.BlockSpec((_BN, D), lambda i: (i, 0)),
            pl.BlockSpec((NC, _BN), lambda i: (0, i)),
        ],
        out_specs=[
            pl.BlockSpec((_BN, D), lambda i: (i, 0)),
            pl.BlockSpec((_BN,), lambda i: (i,)),
        ],
        out_shape=[
            jax.ShapeDtypeStruct((NPAD, D), jnp.float32),
            jax.ShapeDtypeStruct((NPAD,), jnp.float32),
        ],
    )(x, degp)


def _tc_mid(yp, z1, dinv, W1, b1, W2, a):
    D = yp.shape[2]
    H = W1.shape[1]
    DO = W2.shape[1]
    grid = (NPAD // _BN,)

    def body(yp_ref, z1_ref, dinv_ref, W1_ref, b1_ref, W2_ref, a_ref,
             z2_ref):
        dinv = dinv_ref[...][:, None]
        agg = (yp_ref[0] + yp_ref[1] + z1_ref[...]) * dinv
        h = jnp.dot(agg, W1_ref[...], preferred_element_type=jnp.float32)
        h = h + b1_ref[...][None, :]
        av = a_ref[0, 0]
        h = jnp.where(h >= 0.0, h, av * h)
        g = jnp.dot(h, W2_ref[...], preferred_element_type=jnp.float32)
        z2_ref[...] = g * dinv

    return pl.pallas_call(
        body,
        grid=grid,
        in_specs=[
            pl.BlockSpec((NC, _BN, D), lambda i: (0, i, 0)),
            pl.BlockSpec((_BN, D), lambda i: (i, 0)),
            pl.BlockSpec((_BN,), lambda i: (i,)),
            pl.BlockSpec((D, H), lambda i: (0, 0)),
            pl.BlockSpec((H,), lambda i: (0,)),
            pl.BlockSpec((H, DO), lambda i: (0, 0)),
            pl.BlockSpec((1, 1), lambda i: (0, 0), memory_space=pltpu.SMEM),
        ],
        out_specs=pl.BlockSpec((_BN, DO), lambda i: (i, 0)),
        out_shape=jax.ShapeDtypeStruct((NPAD, DO), jnp.float32),
    )(yp, z1, dinv, W1, b1, W2, a)


def _tc_final(yp, z2, dinv, b2, n_out):
    D = yp.shape[2]
    grid = (NPAD // _BN,)

    def body(yp_ref, z2_ref, dinv_ref, b2_ref, out_ref):
        dinv = dinv_ref[...][:, None]
        out_ref[...] = (yp_ref[0] + yp_ref[1] + z2_ref[...]) * dinv \
            + b2_ref[...][None, :]

    return pl.pallas_call(
        body,
        grid=grid,
        in_specs=[
            pl.BlockSpec((NC, _BN, D), lambda i: (0, i, 0)),
            pl.BlockSpec((_BN, D), lambda i: (i, 0)),
            pl.BlockSpec((_BN,), lambda i: (i,)),
            pl.BlockSpec((D,), lambda i: (0,)),
        ],
        out_specs=pl.BlockSpec((_BN, D), lambda i: (i, 0)),
        out_shape=jax.ShapeDtypeStruct((n_out, D), jnp.float32),
    )(yp, z2, dinv, b2)


def kernel(x, edge_index, W1, b1, W2, b2, prelu_a):
    ei5 = jnp.transpose(
        jnp.reshape(edge_index, (2, NW, SUBCH, SCHUNK)),
        (1, 2, 0, 3))
    a = jnp.reshape(prelu_a, (1, 1)).astype(jnp.float32)

    degp = _sc_degree(ei5)
    z1, dinv = _tc_prescale(x, degp)
    y1 = _sc_apply(z1, ei5)
    z2 = _tc_mid(y1, z1, dinv, W1, b1, W2, a)
    y2 = _sc_apply(z2, ei5)
    out = _tc_final(y2, z2, dinv, b2, x.shape[0])
    return out


# speedup vs baseline: 1.0021x; 1.0021x over previous
import functools

import jax
import jax.numpy as jnp
from jax import lax
from jax.experimental import pallas as pl
from jax.experimental.pallas import tpu as pltpu
from jax.experimental.pallas import tpu_sc as plsc

NC = 2
NS = 16
NW = NC * NS

SCHUNK = 50
SUBCH = 200
NPAD = 10240


def _sc_degree(ei5):
    npt = NPAD // NS

    mesh = plsc.VectorSubcoreMesh(core_axis_name="c", subcore_axis_name="s")

    @functools.partial(
        pl.kernel,
        out_type=jax.ShapeDtypeStruct((NC, NPAD), jnp.float32),
        mesh=mesh,
        scratch_types=[
            pltpu.VMEM((SUBCH, 2, SCHUNK), jnp.int32),
            pltpu.VMEM((128,), jnp.float32),
            pltpu.VMEM((npt,), jnp.float32),
            pltpu.VMEM_SHARED((NPAD,), jnp.float32),
            pltpu.SemaphoreType.DMA,
        ],
    )
    def k(ei_hbm, out_hbm, idx_v, ones_v, stage_v, hist_sh, sem):
        c = lax.axis_index("c")
        s = lax.axis_index("s")
        wid = c * NS + s

        def fill_ones(i, _):
            ones_v[pl.ds(i * 16, 16)] = jnp.full((16,), 1.0, jnp.float32)
            return 0

        lax.fori_loop(0, 8, fill_ones, 0)

        def zero_stage(i, _):
            stage_v[pl.ds(i * 16, 16)] = jnp.zeros((16,), jnp.float32)
            return 0

        lax.fori_loop(0, npt // 16, zero_stage, 0)

        pltpu.sync_copy(ei_hbm.at[wid], idx_v)
        pltpu.sync_copy(stage_v, hist_sh.at[pl.ds(s * npt, npt)])
        plsc.subcore_barrier()

        ones = ones_v.at[pl.ds(0, SCHUNK)]

        def fire(q, _):
            pltpu.async_copy(ones, hist_sh.at[idx_v.at[q, 1]], sem, add=True)
            return 0

        lax.fori_loop(0, SUBCH, fire, 0)

        def drain(q, _):
            pltpu.make_async_copy(ones, hist_sh.at[idx_v.at[q, 1]],
                                  sem).wait()
            return 0

        lax.fori_loop(0, SUBCH, drain, 0)
        plsc.subcore_barrier()

        pltpu.sync_copy(hist_sh.at[pl.ds(s * npt, npt)], stage_v)
        pltpu.sync_copy(stage_v, out_hbm.at[c, pl.ds(s * npt, npt)])

    return k(ei5)


def _sc_apply(z, ei5):
    N, D = z.shape
    rpt = NPAD // NS
    RC = 64
    n_rc = rpt // RC
    NB = 4
    NSLOT = 8
    n_outer = SUBCH // NSLOT

    mesh = plsc.VectorSubcoreMesh(core_axis_name="c", subcore_axis_name="s")

    @functools.partial(
        pl.kernel,
        out_type=jax.ShapeDtypeStruct((NC, NPAD, D), jnp.float32),
        mesh=mesh,
        scratch_types=[
            pltpu.VMEM((NSLOT, 2, SCHUNK), jnp.int32),
            [pltpu.VMEM((SCHUNK, D), jnp.float32) for _ in range(NB)],
            pltpu.VMEM((RC, D), jnp.float32),
            pltpu.VMEM_SHARED((NPAD, D), jnp.float32),
            [pltpu.SemaphoreType.DMA for _ in range(NSLOT)],
            [pltpu.SemaphoreType.DMA for _ in range(NB)],
            [pltpu.SemaphoreType.DMA for _ in range(NB)],
            pltpu.SemaphoreType.DMA,
        ],
    )
    def k(z_hbm, ei_hbm, out_hbm, idx_v, rows, stage_v, acc_sh,
          isem, gsem, ssem, osem):
        c = lax.axis_index("c")
        s = lax.axis_index("s")
        wid = c * NS + s

        def load_idx(m, j):
            pltpu.async_copy(ei_hbm.at[wid, m], idx_v.at[j], isem[j])

        def wait_idx(m, j):
            pltpu.make_async_copy(
                ei_hbm.at[wid, m], idx_v.at[j], isem[j]).wait()

        def gather(m, b, j):
            pltpu.async_copy(z_hbm.at[idx_v.at[j, 0]], rows[b], gsem[b])

        def wait_gather(b, j):
            pltpu.make_async_copy(
                z_hbm.at[idx_v.at[j, 0]], rows[b], gsem[b]).wait()

        def scatter(b, j):
            pltpu.async_copy(
                rows[b], acc_sh.at[idx_v.at[j, 1]], ssem[b], add=True)

        def wait_scatter(b, j):
            pltpu.make_async_copy(
                rows[b], acc_sh.at[idx_v.at[j, 1]], ssem[b]).wait()

        for j in range(6):
            load_idx(j, j)

        def zrow(i, _):
            def zcol(j, _):
                stage_v[i, pl.ds(j * 16, 16)] = jnp.zeros((16,), jnp.float32)
                return 0

            lax.fori_loop(0, D // 16, zcol, 0)
            return 0

        lax.fori_loop(0, RC, zrow, 0)

        def zacc(i, _):
            pltpu.async_copy(
                stage_v, acc_sh.at[pl.ds(s * rpt + i * RC, RC)], osem)
            return 0

        lax.fori_loop(0, n_rc, zacc, 0)

        wait_idx(0, 0)
        gather(0, 0, 0)
        wait_idx(1, 1)
        gather(1, 1, 1)

        def zacc_drain(i, _):
            pltpu.make_async_copy(
                stage_v, acc_sh.at[pl.ds(s * rpt + i * RC, RC)], osem).wait()
            return 0

        lax.fori_loop(0, n_rc, zacc_drain, 0)
        plsc.subcore_barrier()

        def outer(i, _):
            for t in range(NSLOT):
                b = t % NB
                wait_gather(b, t)
                scatter(b, t)

                bp = (t - 2) % NB
                jp = (t - 2) % NSLOT
                if t >= 2:
                    wait_scatter(bp, jp)
                else:

                    @pl.when(i > 0)
                    def _():
                        wait_scatter(bp, jp)

                jn = (t + 6) % NSLOT
                if t < 2:
                    load_idx(i * NSLOT + t + 6, jn)
                else:

                    @pl.when(i < n_outer - 1)
                    def _():
                        load_idx(i * NSLOT + t + 6, jn)

                bg = (t + 2) % NB
                jg = (t + 2) % NSLOT
                if t < NSLOT - 2:
                    wait_idx(i * NSLOT + t + 2, jg)
                    gather(i * NSLOT + t + 2, bg, jg)
                else:

                    @pl.when(i < n_outer - 1)
                    def _():
                        wait_idx(i * NSLOT + t + 2, jg)
                        gather(i * NSLOT + t + 2, bg, jg)

            return 0

        lax.fori_loop(0, n_outer, outer, 0)
        wait_scatter((SUBCH - 2) % NB, (SUBCH - 2) % NSLOT)
        wait_scatter((SUBCH - 1) % NB, (SUBCH - 1) % NSLOT)
        plsc.subcore_barrier()

        def ocopy(i, _):
            pltpu.async_copy(
                acc_sh.at[pl.ds(s * rpt + i * RC, RC)],
                out_hbm.at[c, pl.ds(s * rpt + i * RC, RC)], osem)
            return 0

        lax.fori_loop(0, n_rc, ocopy, 0)

        def ocopy_drain(i, _):
            pltpu.make_async_copy(
                acc_sh.at[pl.ds(s * rpt + i * RC, RC)],
                out_hbm.at[c, pl.ds(s * rpt + i * RC, RC)], osem).wait()
            return 0

        lax.fori_loop(0, n_rc, ocopy_drain, 0)

    return k(z, ei5)


_BN = 1024


def _tc_prescale(x, degp):
    N, D = x.shape
    grid = (NPAD // _BN,)

    def body(x_ref, degp_ref, z_ref, dinv_ref):
        deg = degp_ref[0, :] + degp_ref[1, :] + 1.0
        dinv = lax.rsqrt(deg)
        dinv_ref[...] = dinv
        z_ref[...] = x_ref[...] * dinv[:, None]

    return pl.pallas_call(
        body,
        grid=grid,
        in_specs=[
            pl.BlockSpec((_BN, D), lambda i: (i, 0)),
            pl.BlockSpec((NC, _BN), lambda i: (0, i)),
        ],
        out_specs=[
            pl.BlockSpec((_BN, D), lambda i: (i, 0)),
            pl.BlockSpec((_BN,), lambda i: (i,)),
        ],
        out_shape=[
            jax.ShapeDtypeStruct((NPAD, D), jnp.float32),
            jax.ShapeDtypeStruct((NPAD,), jnp.float32),
        ],
    )(x, degp)


def _tc_mid(yp, z1, dinv, W1, b1, W2, a):
    D = yp.shape[2]
    H = W1.shape[1]
    DO = W2.shape[1]
    grid = (NPAD // _BN,)

    def body(yp_ref, z1_ref, dinv_ref, W1_ref, b1_ref, W2_ref, a_ref,
             z2_ref):
        dinv = dinv_ref[...][:, None]
        agg = (yp_ref[0] + yp_ref[1] + z1_ref[...]) * dinv
        h = jnp.dot(agg, W1_ref[...], preferred_element_type=jnp.float32)
        h = h + b1_ref[...][None, :]
        av = a_ref[0, 0]
        h = jnp.where(h >= 0.0, h, av * h)
        g = jnp.dot(h, W2_ref[...], preferred_element_type=jnp.float32)
        z2_ref[...] = g * dinv

    return pl.pallas_call(
        body,
        grid=grid,
        in_specs=[
            pl.BlockSpec((NC, _BN, D), lambda i: (0, i, 0)),
            pl.BlockSpec((_BN, D), lambda i: (i, 0)),
            pl.BlockSpec((_BN,), lambda i: (i,)),
            pl.BlockSpec((D, H), lambda i: (0, 0)),
            pl.BlockSpec((H,), lambda i: (0,)),
            pl.BlockSpec((H, DO), lambda i: (0, 0)),
            pl.BlockSpec((1, 1), lambda i: (0, 0), memory_space=pltpu.SMEM),
        ],
        out_specs=pl.BlockSpec((_BN, DO), lambda i: (i, 0)),
        out_shape=jax.ShapeDtypeStruct((NPAD, DO), jnp.float32),
    )(yp, z1, dinv, W1, b1, W2, a)


def _tc_final(yp, z2, dinv, b2, n_out):
    D = yp.shape[2]
    grid = (NPAD // _BN,)

    def body(yp_ref, z2_ref, dinv_ref, b2_ref, out_ref):
        dinv = dinv_ref[...][:, None]
        out_ref[...] = (yp_ref[0] + yp_ref[1] + z2_ref[...]) * dinv \
            + b2_ref[...][None, :]

    return pl.pallas_call(
        body,
        grid=grid,
        in_specs=[
            pl.BlockSpec((NC, _BN, D), lambda i: (0, i, 0)),
            pl.BlockSpec((_BN, D), lambda i: (i, 0)),
            pl.BlockSpec((_BN,), lambda i: (i,)),
            pl.BlockSpec((D,), lambda i: (0,)),
        ],
        out_specs=pl.BlockSpec((_BN, D), lambda i: (i, 0)),
        out_shape=jax.ShapeDtypeStruct((n_out, D), jnp.float32),
    )(yp, z2, dinv, b2)


def kernel(x, edge_index, W1, b1, W2, b2, prelu_a):
    ei5 = jnp.transpose(
        jnp.reshape(edge_index, (2, NW, SUBCH, SCHUNK)),
        (1, 2, 0, 3))
    a = jnp.reshape(prelu_a, (1, 1)).astype(jnp.float32)

    degp = _sc_degree(ei5)
    z1, dinv = _tc_prescale(x, degp)
    y1 = _sc_apply(z1, ei5)
    z2 = _tc_mid(y1, z1, dinv, W1, b1, W2, a)
    y2 = _sc_apply(z2, ei5)
    out = _tc_final(y2, z2, dinv, b2, x.shape[0])
    return out
